# baseline (speedup 1.0000x reference)
import functools

import jax
import jax.numpy as jnp
from jax.experimental import pallas as pl
from jax.experimental.pallas import tpu as pltpu

_NSIGMA = 3.0
_PATCH = 8
_BN = 1000


def _raster_body(sigma_ref, time_ref, charge_ref, tail_ref, h_ref,
                 out_ref, off_ref):
    sigma = sigma_ref[...]
    tail = tail_ref[...]
    time = time_ref[...]
    charge = charge_ref[...]

    c0 = tail[:, 1:2]
    c1 = tail[:, 0:1]
    c2 = time

    qs = []
    offs = []
    inv_sqrt2 = 0.7071067811865476
    t9 = jax.lax.broadcasted_iota(jnp.int32, (1, _PATCH + 1), 1).astype(jnp.float32)
    for a, c in enumerate((c0, c1, c2)):
        s = sigma[:, a:a + 1]
        h = h_ref[0, a]
        low = c - _NSIGMA * s
        offf = jnp.floor(low / h)
        z = (offf * h - c + t9 * h) * (inv_sqrt2 / s)
        cdf = 0.5 * (1.0 + jax.lax.erf(z))
        qs.append(cdf[:, 1:] - cdf[:, :-1])
        offs.append(offf)

    q0c = qs[0] * charge
    im = jax.lax.broadcasted_iota(jnp.int32, (8, 512), 1)
    ir = jax.lax.broadcasted_iota(jnp.int32, (8, 512), 0)
    e0 = (im // 64 == ir).astype(jnp.float32)
    e1 = ((im // 8) % 8 == ir).astype(jnp.float32)
    e2 = (im % 8 == ir).astype(jnp.float32)
    a0 = jnp.dot(q0c, e0, preferred_element_type=jnp.float32)
    a1 = jnp.dot(qs[1], e1, preferred_element_type=jnp.float32)
    a2 = jnp.dot(qs[2], e2, preferred_element_type=jnp.float32)
    out_ref[...] = a0 * a1 * a2
    off_ref[...] = jnp.concatenate(offs, axis=1).astype(jnp.int32)


def kernel(sigma, time, charge, tail, grid_spacing, velocity):
    n = sigma.shape[0]
    grid = n // _BN
    h = grid_spacing.reshape(1, 3)
    rasters, offsets = pl.pallas_call(
        _raster_body,
        grid=(grid,),
        in_specs=[
            pl.BlockSpec((_BN, 3), lambda i: (i, 0)),
            pl.BlockSpec((_BN, 1), lambda i: (i, 0)),
            pl.BlockSpec((_BN, 1), lambda i: (i, 0)),
            pl.BlockSpec((_BN, 3), lambda i: (i, 0)),
            pl.BlockSpec(memory_space=pltpu.SMEM),
        ],
        out_specs=[
            pl.BlockSpec((_BN, 512), lambda i: (i, 0)),
            pl.BlockSpec((_BN, 3), lambda i: (i, 0)),
        ],
        out_shape=[
            jax.ShapeDtypeStruct((n, 512), jnp.float32),
            jax.ShapeDtypeStruct((n, 3), jnp.int32),
        ],
    )(sigma, time.reshape(n, 1), charge.reshape(n, 1), tail, h)
    return rasters.reshape(n, _PATCH, _PATCH, _PATCH), offsets


# speedup vs baseline: 2.0503x; 2.0503x over previous
import jax
import jax.numpy as jnp
from jax.experimental import pallas as pl
from jax.experimental.pallas import tpu as pltpu

_NSIGMA = 3.0
_PATCH = 8
_BN = 1000
_TINY = 1e-30


def _raster_body(c_ref, s_ref, ch_ref, h_ref, out_ref, off_ref):
    c = c_ref[0]
    s = s_ref[0]
    inv_sqrt2 = 0.7071067811865476

    ir3 = jax.lax.broadcasted_iota(jnp.int32, (3, 1), 0)
    h = jnp.where(ir3 == 0, h_ref[0], jnp.where(ir3 == 1, h_ref[1], h_ref[2]))

    offf = jnp.floor((c - _NSIGMA * s) / h)
    invs = inv_sqrt2 / s
    b0 = (offf * h - c) * invs
    step = h * invs

    cdf_prev = 0.5 * (1.0 + jax.lax.erf(b0))
    rows = []
    for t in range(1, _PATCH + 1):
        cdf = 0.5 * (1.0 + jax.lax.erf(b0 + float(t) * step))
        rows.append(jnp.log(jnp.maximum(cdf - cdf_prev, _TINY)))
        cdf_prev = cdf
    rows.append(jnp.log(jnp.maximum(ch_ref[0], _TINY)))
    lq = jnp.concatenate(rows, axis=0)

    im = jax.lax.broadcasted_iota(jnp.int32, (25, 512), 1)
    ir = jax.lax.broadcasted_iota(jnp.int32, (25, 512), 0)
    t_r, a_r = ir // 3, ir % 3
    sel = (((a_r == 0) & (im // 64 == t_r))
           | ((a_r == 1) & ((im // 8) % 8 == t_r))
           | ((a_r == 2) & (im % 8 == t_r))
           | (ir == 24)).astype(jnp.float32)
    acc = jax.lax.dot_general(lq, sel, (((0,), (0,)), ((), ())),
                              preferred_element_type=jnp.float32)
    out_ref[...] = jnp.exp(acc)
    off_ref[0] = offf.astype(jnp.int32)


def kernel(sigma, time, charge, tail, grid_spacing, velocity):
    n = sigma.shape[0]
    grid = n // _BN
    c_t = jnp.stack([tail[:, 1], tail[:, 0], time]).reshape(3, grid, _BN)
    c_t = c_t.transpose(1, 0, 2)
    s_t = sigma.T.reshape(3, grid, _BN).transpose(1, 0, 2)
    ch_t = charge.reshape(grid, 1, _BN)
    rasters, offsets_t = pl.pallas_call(
        _raster_body,
        grid=(grid,),
        in_specs=[
            pl.BlockSpec((1, 3, _BN), lambda i: (i, 0, 0)),
            pl.BlockSpec((1, 3, _BN), lambda i: (i, 0, 0)),
            pl.BlockSpec((1, 1, _BN), lambda i: (i, 0, 0)),
            pl.BlockSpec(memory_space=pltpu.SMEM),
        ],
        out_specs=[
            pl.BlockSpec((_BN, 512), lambda i: (i, 0)),
            pl.BlockSpec((1, 3, _BN), lambda i: (i, 0, 0)),
        ],
        out_shape=[
            jax.ShapeDtypeStruct((n, 512), jnp.float32),
            jax.ShapeDtypeStruct((grid, 3, _BN), jnp.int32),
        ],
    )(c_t, s_t, ch_t, grid_spacing)
    offsets = offsets_t.transpose(1, 0, 2).reshape(3, n).T
    return rasters.reshape(n, _PATCH, _PATCH, _PATCH), offsets
